# Initial kernel scaffold; baseline (speedup 1.0000x reference)
import functools

import jax
import jax.numpy as jnp
from jax import lax
from jax.experimental import pallas as pl
from jax.experimental.pallas import tpu as pltpu
from jax.experimental.pallas import tpu_sc as plsc

N_NODES = 10000
N_EDGES = 320000
DIM = 128
W16 = 16

NC = 2
NS = 16
NW = NC * NS
EPW = N_EDGES // NW
CHUNK = 80
NCHUNKS = EPW // CHUNK
ROWS_PER_SUB = N_NODES // NS

_sc_mesh = plsc.VectorSubcoreMesh(core_axis_name="c", subcore_axis_name="s")


def _make_sc_agg(width):
  @functools.partial(
      pl.kernel,
      out_type=jax.ShapeDtypeStruct((NC, N_NODES, width), jnp.float32),
      mesh=_sc_mesh,
      scratch_types=[
          pltpu.VMEM((CHUNK,), jnp.int32),
          pltpu.VMEM((CHUNK,), jnp.int32),
          pltpu.VMEM((CHUNK, width), jnp.float32),
          pltpu.VMEM_SHARED((N_NODES, width), jnp.float32),
          pltpu.SemaphoreType.DMA,
      ],
  )
  def k(src_hbm, dst_hbm, table_hbm, zeros_hbm, out_hbm,
        src_v, dst_v, rows_v, acc_sh, sem):
    c = lax.axis_index("c")
    s = lax.axis_index("s")
    wid = c * NS + s
    row0 = s * ROWS_PER_SUB
    pltpu.sync_copy(zeros_hbm.at[pl.ds(row0, ROWS_PER_SUB)],
                    acc_sh.at[pl.ds(row0, ROWS_PER_SUB)])
    plsc.subcore_barrier()

    base0 = wid * EPW

    @pl.loop(0, NCHUNKS)
    def _(i):
      base = base0 + i * CHUNK
      pltpu.sync_copy(src_hbm.at[pl.ds(base, CHUNK)], src_v)
      pltpu.sync_copy(dst_hbm.at[pl.ds(base, CHUNK)], dst_v)
      pltpu.async_copy(table_hbm.at[src_v], rows_v, sem).wait()
      pltpu.sync_copy(rows_v, acc_sh.at[dst_v], add=True)

    plsc.subcore_barrier()
    pltpu.sync_copy(acc_sh.at[pl.ds(row0, ROWS_PER_SUB)],
                    out_hbm.at[c, pl.ds(row0, ROWS_PER_SUB)])

  return k


@functools.partial(
    pl.kernel,
    out_type=jax.ShapeDtypeStruct((NC, N_NODES, W16), jnp.float32),
    mesh=_sc_mesh,
    scratch_types=[
        pltpu.VMEM((CHUNK,), jnp.int32),
        pltpu.VMEM((CHUNK, W16), jnp.float32),
        pltpu.VMEM_SHARED((N_NODES, W16), jnp.float32),
    ],
)
def _sc_degree(dst_hbm, ones_hbm, zeros_hbm, out_hbm, dst_v, ones_v, acc_sh):
  c = lax.axis_index("c")
  s = lax.axis_index("s")
  wid = c * NS + s
  row0 = s * ROWS_PER_SUB
  pltpu.sync_copy(zeros_hbm.at[pl.ds(row0, ROWS_PER_SUB)],
                  acc_sh.at[pl.ds(row0, ROWS_PER_SUB)])
  pltpu.sync_copy(ones_hbm, ones_v)
  plsc.subcore_barrier()

  base0 = wid * EPW

  @pl.loop(0, NCHUNKS)
  def _(i):
    pltpu.sync_copy(dst_hbm.at[pl.ds(base0 + i * CHUNK, CHUNK)], dst_v)
    pltpu.sync_copy(ones_v, acc_sh.at[dst_v], add=True)

  plsc.subcore_barrier()
  pltpu.sync_copy(acc_sh.at[pl.ds(row0, ROWS_PER_SUB)],
                  out_hbm.at[c, pl.ds(row0, ROWS_PER_SUB)])


_sc_agg128 = _make_sc_agg(DIM)
_sc_agg16 = _make_sc_agg(W16)


_RB = 1000
_GRID = N_NODES // _RB


def _tc_matmul_body(x_ref, w_ref, o_ref):
  o_ref[...] = jnp.dot(x_ref[...], w_ref[...],
                       preferred_element_type=jnp.float32)


def _tc_matmul(x, w):
  return pl.pallas_call(
      _tc_matmul_body,
      grid=(_GRID,),
      in_specs=[
          pl.BlockSpec((_RB, DIM), lambda i: (i, 0)),
          pl.BlockSpec((DIM, DIM), lambda i: (0, 0)),
      ],
      out_specs=pl.BlockSpec((_RB, DIM), lambda i: (i, 0)),
      out_shape=jax.ShapeDtypeStruct((N_NODES, DIM), jnp.float32),
  )(x, w)


def _tc_scale_body(deg_ref, h_ref, h1p_ref, dinv_ref):
  deg = deg_ref[0, :, 0] + deg_ref[1, :, 0] + 1.0
  dinv = lax.rsqrt(deg)
  h1p_ref[...] = h_ref[...] * dinv[:, None]
  dinv_ref[...] = dinv[:, None]


def _tc_scale(deg_parts, h):
  return pl.pallas_call(
      _tc_scale_body,
      grid=(_GRID,),
      in_specs=[
          pl.BlockSpec((NC, _RB, W16), lambda i: (0, i, 0)),
          pl.BlockSpec((_RB, DIM), lambda i: (i, 0)),
      ],
      out_specs=[
          pl.BlockSpec((_RB, DIM), lambda i: (i, 0)),
          pl.BlockSpec((_RB, 1), lambda i: (i, 0)),
      ],
      out_shape=[
          jax.ShapeDtypeStruct((N_NODES, DIM), jnp.float32),
          jax.ShapeDtypeStruct((N_NODES, 1), jnp.float32),
      ],
  )(deg_parts, h)


def _tc_layer1_body(p_ref, h1p_ref, dinv_ref, b1_ref, w2_ref, h2p_ref):
  dinv = dinv_ref[...]
  agg = p_ref[0] + p_ref[1] + h1p_ref[...]
  z1 = jnp.maximum(dinv * agg + b1_ref[...], 0.0)
  h2 = jnp.dot(z1, w2_ref[...], preferred_element_type=jnp.float32)
  col = lax.broadcasted_iota(jnp.int32, (_RB, W16), 1)
  h2p_ref[...] = jnp.where(col == 0, dinv * h2, 0.0)


def _tc_layer1(parts, h1p, dinv, b1, w2):
  return pl.pallas_call(
      _tc_layer1_body,
      grid=(_GRID,),
      in_specs=[
          pl.BlockSpec((NC, _RB, DIM), lambda i: (0, i, 0)),
          pl.BlockSpec((_RB, DIM), lambda i: (i, 0)),
          pl.BlockSpec((_RB, 1), lambda i: (i, 0)),
          pl.BlockSpec((1, DIM), lambda i: (0, 0)),
          pl.BlockSpec((DIM, 1), lambda i: (0, 0)),
      ],
      out_specs=pl.BlockSpec((_RB, W16), lambda i: (i, 0)),
      out_shape=jax.ShapeDtypeStruct((N_NODES, W16), jnp.float32),
  )(parts, h1p, dinv, b1, w2)


def _tc_final_body(q_ref, h2p_ref, dinv_ref, b2_ref, o_ref):
  tot = q_ref[0] + q_ref[1] + h2p_ref[...]
  o_ref[...] = dinv_ref[...] * tot[:, 0:1] + b2_ref[...]


def _tc_final(qparts, h2p, dinv, b2):
  return pl.pallas_call(
      _tc_final_body,
      grid=(_GRID,),
      in_specs=[
          pl.BlockSpec((NC, _RB, W16), lambda i: (0, i, 0)),
          pl.BlockSpec((_RB, W16), lambda i: (i, 0)),
          pl.BlockSpec((_RB, 1), lambda i: (i, 0)),
          pl.BlockSpec((1, 1), lambda i: (0, 0)),
      ],
      out_specs=pl.BlockSpec((_RB, 1), lambda i: (i, 0)),
      out_shape=jax.ShapeDtypeStruct((N_NODES, 1), jnp.float32),
  )(qparts, h2p, dinv, b2)


@jax.jit
def kernel(x, edge_index, W1, b1, W2, b2):
  src = edge_index[0].astype(jnp.int32)
  dst = edge_index[1].astype(jnp.int32)

  zeros16 = jnp.zeros((N_NODES, W16), jnp.float32)
  zeros128 = jnp.zeros((N_NODES, DIM), jnp.float32)
  ones16 = jnp.ones((CHUNK, W16), jnp.float32)

  deg_parts = _sc_degree(dst, ones16, zeros16)
  h = _tc_matmul(x, W1)

  h1p, dinv = _tc_scale(deg_parts, h)

  parts = _sc_agg128(src, dst, h1p, zeros128)

  h2p = _tc_layer1(parts, h1p, dinv, b1.reshape(1, DIM), W2)

  qparts = _sc_agg16(src, dst, h2p, zeros16)

  out = _tc_final(qparts, h2p, dinv, b2.reshape(1, 1))
  return out


# speedup vs baseline: 25.3776x; 25.3776x over previous
import dataclasses
import functools

import jax
import jax.numpy as jnp
from jax import lax
from jax.experimental import pallas as pl
from jax.experimental.pallas import tpu as pltpu
from jax.experimental.pallas import tpu_sc as plsc

N_NODES = 10000
N_PAD = 10240
N_EDGES = 320000
DIM = 128

NC = 2
NS = 16
NL = 16
NW = NC * NS
EPW = N_EDGES // NW
CHUNK = 80
NCHUNKS = EPW // CHUNK
ROWS_PER_SUB = N_PAD // NS

_sc_mesh = plsc.VectorSubcoreMesh(core_axis_name="c", subcore_axis_name="s")

_sc_cp = pltpu.CompilerParams()
if "needs_layout_passes" in pltpu.CompilerParams.__dataclass_fields__:
  _sc_cp = dataclasses.replace(_sc_cp, needs_layout_passes=False)


@functools.partial(
    pl.kernel,
    out_type=jax.ShapeDtypeStruct((NC, N_PAD, DIM), jnp.float32),
    mesh=_sc_mesh,
    scratch_types=[
        pltpu.VMEM((CHUNK,), jnp.int32),
        pltpu.VMEM((CHUNK,), jnp.int32),
        pltpu.VMEM((CHUNK, DIM), jnp.float32),
        pltpu.VMEM_SHARED((N_PAD, DIM), jnp.float32),
        pltpu.SemaphoreType.DMA,
    ],
)
def _sc_agg128(src_hbm, dst_hbm, table_hbm, zeros_hbm, out_hbm,
               src_v, dst_v, rows_v, acc_sh, sem):
  c = lax.axis_index("c")
  s = lax.axis_index("s")
  wid = c * NS + s
  row0 = s * ROWS_PER_SUB
  pltpu.sync_copy(zeros_hbm.at[pl.ds(row0, ROWS_PER_SUB)],
                  acc_sh.at[pl.ds(row0, ROWS_PER_SUB)])
  plsc.subcore_barrier()

  base0 = wid * EPW

  @pl.loop(0, NCHUNKS)
  def _(i):
    base = base0 + i * CHUNK
    pltpu.sync_copy(src_hbm.at[pl.ds(base, CHUNK)], src_v)
    pltpu.sync_copy(dst_hbm.at[pl.ds(base, CHUNK)], dst_v)
    pltpu.async_copy(table_hbm.at[src_v], rows_v, sem).wait()
    pltpu.sync_copy(rows_v, acc_sh.at[dst_v], add=True)

  plsc.subcore_barrier()
  pltpu.sync_copy(acc_sh.at[pl.ds(row0, ROWS_PER_SUB)],
                  out_hbm.at[c, pl.ds(row0, ROWS_PER_SUB)])


@functools.partial(
    pl.kernel,
    out_type=jax.ShapeDtypeStruct((NW, N_PAD), jnp.float32),
    mesh=_sc_mesh,
    compiler_params=_sc_cp,
    scratch_types=[
        pltpu.VMEM((EPW,), jnp.int32),
        pltpu.VMEM((N_PAD,), jnp.float32),
    ],
)
def _sc_degree(dst_hbm, out_hbm, dst_v, acc_v):
  c = lax.axis_index("c")
  s = lax.axis_index("s")
  wid = c * NS + s

  zeros = jnp.zeros((NL,), jnp.float32)

  @pl.loop(0, N_PAD // NL)
  def _(j):
    acc_v[pl.ds(j * NL, NL)] = zeros

  pltpu.sync_copy(dst_hbm.at[pl.ds(wid * EPW, EPW)], dst_v)

  ones = jnp.ones((NL,), jnp.float32)

  @pl.loop(0, EPW // NL)
  def _(i):
    idx = dst_v[pl.ds(i * NL, NL)]
    plsc.addupdate_scatter(acc_v, [idx], ones)

  pltpu.sync_copy(acc_v, out_hbm.at[wid])


@functools.partial(
    pl.kernel,
    out_type=jax.ShapeDtypeStruct((NW, N_PAD), jnp.float32),
    mesh=_sc_mesh,
    compiler_params=_sc_cp,
    scratch_types=[
        pltpu.VMEM((EPW,), jnp.int32),
        pltpu.VMEM((EPW,), jnp.int32),
        pltpu.VMEM((N_PAD,), jnp.float32),
        pltpu.VMEM((N_PAD,), jnp.float32),
    ],
)
def _sc_agg_scalar(src_hbm, dst_hbm, table_hbm, out_hbm,
                   src_v, dst_v, tab_v, acc_v):
  c = lax.axis_index("c")
  s = lax.axis_index("s")
  wid = c * NS + s

  zeros = jnp.zeros((NL,), jnp.float32)

  @pl.loop(0, N_PAD // NL)
  def _(j):
    acc_v[pl.ds(j * NL, NL)] = zeros

  pltpu.sync_copy(table_hbm, tab_v)
  pltpu.sync_copy(src_hbm.at[pl.ds(wid * EPW, EPW)], src_v)
  pltpu.sync_copy(dst_hbm.at[pl.ds(wid * EPW, EPW)], dst_v)

  @pl.loop(0, EPW // NL)
  def _(i):
    si = src_v[pl.ds(i * NL, NL)]
    di = dst_v[pl.ds(i * NL, NL)]
    val = plsc.load_gather(tab_v, [si])
    plsc.addupdate_scatter(acc_v, [di], val)

  pltpu.sync_copy(acc_v, out_hbm.at[wid])


_RB = 1024
_GRID = N_PAD // _RB


def _tc_matmul_body(x_ref, w_ref, o_ref):
  o_ref[...] = jnp.dot(x_ref[...], w_ref[...],
                       preferred_element_type=jnp.float32)


def _tc_matmul(x, w):
  return pl.pallas_call(
      _tc_matmul_body,
      grid=(_GRID,),
      in_specs=[
          pl.BlockSpec((_RB, DIM), lambda i: (i, 0)),
          pl.BlockSpec((DIM, DIM), lambda i: (0, 0)),
      ],
      out_specs=pl.BlockSpec((_RB, DIM), lambda i: (i, 0)),
      out_shape=jax.ShapeDtypeStruct((N_PAD, DIM), jnp.float32),
  )(x, w)


def _tc_scale_body(deg_ref, h_ref, h1p_ref, dinv_ref):
  deg = jnp.sum(deg_ref[...], axis=0) + 1.0
  dinv = lax.rsqrt(deg)
  h1p_ref[...] = h_ref[...] * dinv[:, None]
  dinv_ref[...] = dinv[:, None]


def _tc_scale(deg_parts, h):
  return pl.pallas_call(
      _tc_scale_body,
      grid=(_GRID,),
      in_specs=[
          pl.BlockSpec((NW, _RB), lambda i: (0, i)),
          pl.BlockSpec((_RB, DIM), lambda i: (i, 0)),
      ],
      out_specs=[
          pl.BlockSpec((_RB, DIM), lambda i: (i, 0)),
          pl.BlockSpec((_RB, 1), lambda i: (i, 0)),
      ],
      out_shape=[
          jax.ShapeDtypeStruct((N_PAD, DIM), jnp.float32),
          jax.ShapeDtypeStruct((N_PAD, 1), jnp.float32),
      ],
  )(deg_parts, h)


def _tc_layer1_body(p_ref, h1p_ref, dinv_ref, b1_ref, w2_ref, h2s_ref):
  dinv = dinv_ref[...]
  agg = p_ref[0] + p_ref[1] + h1p_ref[...]
  z1 = jnp.maximum(dinv * agg + b1_ref[...], 0.0)
  h2 = jnp.dot(z1, w2_ref[...], preferred_element_type=jnp.float32)
  h2s_ref[...] = dinv * h2


def _tc_layer1(parts, h1p, dinv, b1, w2):
  return pl.pallas_call(
      _tc_layer1_body,
      grid=(_GRID,),
      in_specs=[
          pl.BlockSpec((NC, _RB, DIM), lambda i: (0, i, 0)),
          pl.BlockSpec((_RB, DIM), lambda i: (i, 0)),
          pl.BlockSpec((_RB, 1), lambda i: (i, 0)),
          pl.BlockSpec((1, DIM), lambda i: (0, 0)),
          pl.BlockSpec((DIM, 1), lambda i: (0, 0)),
      ],
      out_specs=pl.BlockSpec((_RB, 1), lambda i: (i, 0)),
      out_shape=jax.ShapeDtypeStruct((N_PAD, 1), jnp.float32),
  )(parts, h1p, dinv, b1, w2)


def _tc_final_body(q_ref, h2s_ref, dinv_ref, b2_ref, o_ref):
  qsum = jnp.sum(q_ref[...], axis=0)[:, None]
  o_ref[...] = dinv_ref[...] * (qsum + h2s_ref[...]) + b2_ref[...]


def _tc_final(qparts, h2s, dinv, b2):
  return pl.pallas_call(
      _tc_final_body,
      grid=(_GRID,),
      in_specs=[
          pl.BlockSpec((NW, _RB), lambda i: (0, i)),
          pl.BlockSpec((_RB, 1), lambda i: (i, 0)),
          pl.BlockSpec((_RB, 1), lambda i: (i, 0)),
          pl.BlockSpec((1, 1), lambda i: (0, 0)),
      ],
      out_specs=pl.BlockSpec((_RB, 1), lambda i: (i, 0)),
      out_shape=jax.ShapeDtypeStruct((N_PAD, 1), jnp.float32),
  )(qparts, h2s, dinv, b2)


@jax.jit
def kernel(x, edge_index, W1, b1, W2, b2):
  src = edge_index[0].astype(jnp.int32)
  dst = edge_index[1].astype(jnp.int32)
  x = jnp.pad(x, ((0, N_PAD - N_NODES), (0, 0)))

  zeros128 = jnp.zeros((N_PAD, DIM), jnp.float32)

  deg_parts = _sc_degree(dst)
  h = _tc_matmul(x, W1)

  h1p, dinv = _tc_scale(deg_parts, h)

  parts = _sc_agg128(src, dst, h1p, zeros128)

  h2s = _tc_layer1(parts, h1p, dinv, b1.reshape(1, DIM), W2)

  qparts = _sc_agg_scalar(src, dst, h2s.reshape(N_PAD))

  out = _tc_final(qparts, h2s, dinv, b2.reshape(1, 1))
  return out[:N_NODES]


# speedup vs baseline: 43.8071x; 1.7262x over previous
import dataclasses
import functools

import jax
import jax.numpy as jnp
from jax import lax
from jax.experimental import pallas as pl
from jax.experimental.pallas import tpu as pltpu
from jax.experimental.pallas import tpu_sc as plsc

N_NODES = 10000
N_PAD = 10240
N_EDGES = 320000
DIM = 128

NC = 2
NS = 16
NL = 16
NW = NC * NS
EPW = N_EDGES // NW
CHUNK = 80
NCHUNKS = EPW // CHUNK
ROWS_PER_SUB = N_PAD // NS

_sc_mesh = plsc.VectorSubcoreMesh(core_axis_name="c", subcore_axis_name="s")

_sc_cp = pltpu.CompilerParams()
if "needs_layout_passes" in pltpu.CompilerParams.__dataclass_fields__:
  _sc_cp = dataclasses.replace(_sc_cp, needs_layout_passes=False)


NBUF = 4
NGROUPS = NCHUNKS // NBUF
NTAIL = NCHUNKS % NBUF


@functools.partial(
    pl.kernel,
    out_type=jax.ShapeDtypeStruct((NC, N_PAD, DIM), jnp.float32),
    mesh=_sc_mesh,
    scratch_types=(
        [pltpu.VMEM((CHUNK,), jnp.int32) for _ in range(2 * NBUF)]
        + [pltpu.VMEM((CHUNK, DIM), jnp.float32) for _ in range(NBUF)]
        + [
            pltpu.VMEM_SHARED((N_PAD, DIM), jnp.float32),
            pltpu.SemaphoreType.DMA,
            pltpu.SemaphoreType.DMA,
        ]
    ),
)
def _sc_agg128(src_hbm, dst_hbm, table_hbm, zeros_hbm, out_hbm, *refs):
  srcs_v = refs[0:NBUF]
  dsts_v = refs[NBUF:2 * NBUF]
  rows_v = refs[2 * NBUF:3 * NBUF]
  acc_sh, isem, gsem = refs[3 * NBUF:]
  c = lax.axis_index("c")
  s = lax.axis_index("s")
  wid = c * NS + s
  row0 = s * ROWS_PER_SUB
  pltpu.sync_copy(zeros_hbm.at[pl.ds(row0, ROWS_PER_SUB)],
                  acc_sh.at[pl.ds(row0, ROWS_PER_SUB)])
  plsc.subcore_barrier()

  base0 = wid * EPW

  def do_group(base, nbuf):
    icopies = []
    for b in range(nbuf):
      icopies.append(
          (pltpu.async_copy(src_hbm.at[pl.ds(base + b * CHUNK, CHUNK)],
                            srcs_v[b], isem),
           pltpu.async_copy(dst_hbm.at[pl.ds(base + b * CHUNK, CHUNK)],
                            dsts_v[b], isem)))
    gathers = []
    for b in range(nbuf):
      icopies[b][0].wait()
      icopies[b][1].wait()
      gathers.append(pltpu.async_copy(
          table_hbm.at[srcs_v[b]], rows_v[b], gsem))
    for b in range(nbuf):
      gathers[b].wait()
      pltpu.sync_copy(rows_v[b], acc_sh.at[dsts_v[b]], add=True)

  @pl.loop(0, NGROUPS)
  def _(j):
    do_group(base0 + j * (NBUF * CHUNK), NBUF)

  for t in range(NTAIL):
    do_group(base0 + (NGROUPS * NBUF + t) * CHUNK, 1)

  plsc.subcore_barrier()
  pltpu.sync_copy(acc_sh.at[pl.ds(row0, ROWS_PER_SUB)],
                  out_hbm.at[c, pl.ds(row0, ROWS_PER_SUB)])


@functools.partial(
    pl.kernel,
    out_type=jax.ShapeDtypeStruct((NW, N_PAD), jnp.float32),
    mesh=_sc_mesh,
    compiler_params=_sc_cp,
    scratch_types=[
        pltpu.VMEM((EPW,), jnp.int32),
        pltpu.VMEM((N_PAD,), jnp.float32),
    ],
)
def _sc_degree(dst_hbm, out_hbm, dst_v, acc_v):
  c = lax.axis_index("c")
  s = lax.axis_index("s")
  wid = c * NS + s

  zeros = jnp.zeros((NL,), jnp.float32)

  @pl.loop(0, N_PAD // NL)
  def _(j):
    acc_v[pl.ds(j * NL, NL)] = zeros

  pltpu.sync_copy(dst_hbm.at[pl.ds(wid * EPW, EPW)], dst_v)

  ones = jnp.ones((NL,), jnp.float32)

  @pl.loop(0, EPW // NL)
  def _(i):
    idx = dst_v[pl.ds(i * NL, NL)]
    plsc.addupdate_scatter(acc_v, [idx], ones)

  pltpu.sync_copy(acc_v, out_hbm.at[wid])


@functools.partial(
    pl.kernel,
    out_type=jax.ShapeDtypeStruct((NW, N_PAD), jnp.float32),
    mesh=_sc_mesh,
    compiler_params=_sc_cp,
    scratch_types=[
        pltpu.VMEM((EPW,), jnp.int32),
        pltpu.VMEM((EPW,), jnp.int32),
        pltpu.VMEM((N_PAD,), jnp.float32),
        pltpu.VMEM((N_PAD,), jnp.float32),
    ],
)
def _sc_agg_scalar(src_hbm, dst_hbm, table_hbm, out_hbm,
                   src_v, dst_v, tab_v, acc_v):
  c = lax.axis_index("c")
  s = lax.axis_index("s")
  wid = c * NS + s

  zeros = jnp.zeros((NL,), jnp.float32)

  @pl.loop(0, N_PAD // NL)
  def _(j):
    acc_v[pl.ds(j * NL, NL)] = zeros

  pltpu.sync_copy(table_hbm, tab_v)
  pltpu.sync_copy(src_hbm.at[pl.ds(wid * EPW, EPW)], src_v)
  pltpu.sync_copy(dst_hbm.at[pl.ds(wid * EPW, EPW)], dst_v)

  @pl.loop(0, EPW // NL)
  def _(i):
    si = src_v[pl.ds(i * NL, NL)]
    di = dst_v[pl.ds(i * NL, NL)]
    val = plsc.load_gather(tab_v, [si])
    plsc.addupdate_scatter(acc_v, [di], val)

  pltpu.sync_copy(acc_v, out_hbm.at[wid])


_RB = 1024
_GRID = N_PAD // _RB


def _tc_matmul_body(x_ref, w_ref, o_ref):
  o_ref[...] = jnp.dot(x_ref[...], w_ref[...],
                       preferred_element_type=jnp.float32)


def _tc_matmul(x, w):
  return pl.pallas_call(
      _tc_matmul_body,
      grid=(_GRID,),
      in_specs=[
          pl.BlockSpec((_RB, DIM), lambda i: (i, 0)),
          pl.BlockSpec((DIM, DIM), lambda i: (0, 0)),
      ],
      out_specs=pl.BlockSpec((_RB, DIM), lambda i: (i, 0)),
      out_shape=jax.ShapeDtypeStruct((N_PAD, DIM), jnp.float32),
  )(x, w)


def _tc_scale_body(deg_ref, h_ref, h1p_ref, dinv_ref):
  deg = jnp.sum(deg_ref[...], axis=0) + 1.0
  dinv = lax.rsqrt(deg)
  h1p_ref[...] = h_ref[...] * dinv[:, None]
  dinv_ref[...] = dinv[:, None]


def _tc_scale(deg_parts, h):
  return pl.pallas_call(
      _tc_scale_body,
      grid=(_GRID,),
      in_specs=[
          pl.BlockSpec((NW, _RB), lambda i: (0, i)),
          pl.BlockSpec((_RB, DIM), lambda i: (i, 0)),
      ],
      out_specs=[
          pl.BlockSpec((_RB, DIM), lambda i: (i, 0)),
          pl.BlockSpec((_RB, 1), lambda i: (i, 0)),
      ],
      out_shape=[
          jax.ShapeDtypeStruct((N_PAD, DIM), jnp.float32),
          jax.ShapeDtypeStruct((N_PAD, 1), jnp.float32),
      ],
  )(deg_parts, h)


def _tc_layer1_body(p_ref, h1p_ref, dinv_ref, b1_ref, w2_ref, h2s_ref):
  dinv = dinv_ref[...]
  agg = p_ref[0] + p_ref[1] + h1p_ref[...]
  z1 = jnp.maximum(dinv * agg + b1_ref[...], 0.0)
  h2 = jnp.dot(z1, w2_ref[...], preferred_element_type=jnp.float32)
  h2s_ref[...] = dinv * h2


def _tc_layer1(parts, h1p, dinv, b1, w2):
  return pl.pallas_call(
      _tc_layer1_body,
      grid=(_GRID,),
      in_specs=[
          pl.BlockSpec((NC, _RB, DIM), lambda i: (0, i, 0)),
          pl.BlockSpec((_RB, DIM), lambda i: (i, 0)),
          pl.BlockSpec((_RB, 1), lambda i: (i, 0)),
          pl.BlockSpec((1, DIM), lambda i: (0, 0)),
          pl.BlockSpec((DIM, 1), lambda i: (0, 0)),
      ],
      out_specs=pl.BlockSpec((_RB, 1), lambda i: (i, 0)),
      out_shape=jax.ShapeDtypeStruct((N_PAD, 1), jnp.float32),
  )(parts, h1p, dinv, b1, w2)


def _tc_final_body(q_ref, h2s_ref, dinv_ref, b2_ref, o_ref):
  qsum = jnp.sum(q_ref[...], axis=0)[:, None]
  o_ref[...] = dinv_ref[...] * (qsum + h2s_ref[...]) + b2_ref[...]


def _tc_final(qparts, h2s, dinv, b2):
  return pl.pallas_call(
      _tc_final_body,
      grid=(_GRID,),
      in_specs=[
          pl.BlockSpec((NW, _RB), lambda i: (0, i)),
          pl.BlockSpec((_RB, 1), lambda i: (i, 0)),
          pl.BlockSpec((_RB, 1), lambda i: (i, 0)),
          pl.BlockSpec((1, 1), lambda i: (0, 0)),
      ],
      out_specs=pl.BlockSpec((_RB, 1), lambda i: (i, 0)),
      out_shape=jax.ShapeDtypeStruct((N_PAD, 1), jnp.float32),
  )(qparts, h2s, dinv, b2)


@jax.jit
def kernel(x, edge_index, W1, b1, W2, b2):
  src = edge_index[0].astype(jnp.int32)
  dst = edge_index[1].astype(jnp.int32)
  x = jnp.pad(x, ((0, N_PAD - N_NODES), (0, 0)))

  zeros128 = jnp.zeros((N_PAD, DIM), jnp.float32)

  deg_parts = _sc_degree(dst)
  h = _tc_matmul(x, W1)

  h1p, dinv = _tc_scale(deg_parts, h)

  parts = _sc_agg128(src, dst, h1p, zeros128)

  h2s = _tc_layer1(parts, h1p, dinv, b1.reshape(1, DIM), W2)

  qparts = _sc_agg_scalar(src, dst, h2s.reshape(N_PAD))

  out = _tc_final(qparts, h2s, dinv, b2.reshape(1, 1))
  return out[:N_NODES]


# speedup vs baseline: 44.1536x; 1.0079x over previous
import dataclasses
import functools

import jax
import jax.numpy as jnp
from jax import lax
from jax.experimental import pallas as pl
from jax.experimental.pallas import tpu as pltpu
from jax.experimental.pallas import tpu_sc as plsc

N_NODES = 10000
N_PAD = 10240
N_EDGES = 320000
DIM = 128

NC = 2
NS = 16
NL = 16
NW = NC * NS
EPW = N_EDGES // NW
CHUNK = 80
NCHUNKS = EPW // CHUNK
REM = EPW % CHUNK
ROWS_PER_SUB = N_PAD // NS

_sc_mesh = plsc.VectorSubcoreMesh(core_axis_name="c", subcore_axis_name="s")

_sc_cp = pltpu.CompilerParams()
if "needs_layout_passes" in pltpu.CompilerParams.__dataclass_fields__:
  _sc_cp = dataclasses.replace(_sc_cp, needs_layout_passes=False)


NBUF = 4
NGROUPS = NCHUNKS // NBUF
NTAIL = NCHUNKS % NBUF


@functools.partial(
    pl.kernel,
    out_type=jax.ShapeDtypeStruct((NC, N_PAD, DIM), jnp.float32),
    mesh=_sc_mesh,
    scratch_types=(
        [pltpu.VMEM((CHUNK,), jnp.int32) for _ in range(2 * NBUF)]
        + [pltpu.VMEM((CHUNK, DIM), jnp.float32) for _ in range(NBUF)]
        + ([pltpu.VMEM((REM,), jnp.int32),
            pltpu.VMEM((REM,), jnp.int32),
            pltpu.VMEM((REM, DIM), jnp.float32)] if REM else [])
        + [
            pltpu.VMEM_SHARED((N_PAD, DIM), jnp.float32),
            pltpu.SemaphoreType.DMA,
            pltpu.SemaphoreType.DMA,
            pltpu.SemaphoreType.DMA,
        ]
    ),
)
def _sc_agg128(src_hbm, dst_hbm, table_hbm, zeros_hbm, out_hbm, *refs):
  srcs_v = refs[0:NBUF]
  dsts_v = refs[NBUF:2 * NBUF]
  rows_v = refs[2 * NBUF:3 * NBUF]
  if REM:
    srcT, dstT, rowsT = refs[3 * NBUF:3 * NBUF + 3]
  acc_sh, isem, gsem, ssem = refs[-4:]
  c = lax.axis_index("c")
  s = lax.axis_index("s")
  wid = c * NS + s
  row0 = s * ROWS_PER_SUB
  pltpu.sync_copy(zeros_hbm.at[pl.ds(row0, ROWS_PER_SUB)],
                  acc_sh.at[pl.ds(row0, ROWS_PER_SUB)])
  plsc.subcore_barrier()

  base0 = wid * EPW

  def do_group(base, nbuf):
    icopies = []
    for b in range(nbuf):
      icopies.append(
          (pltpu.async_copy(src_hbm.at[pl.ds(base + b * CHUNK, CHUNK)],
                            srcs_v[b], isem),
           pltpu.async_copy(dst_hbm.at[pl.ds(base + b * CHUNK, CHUNK)],
                            dsts_v[b], isem)))
    gathers = []
    for b in range(nbuf):
      icopies[b][0].wait()
      icopies[b][1].wait()
      gathers.append(pltpu.async_copy(
          table_hbm.at[srcs_v[b]], rows_v[b], gsem))
    scatters = []
    for b in range(nbuf):
      gathers[b].wait()
      scatters.append(pltpu.async_copy(
          rows_v[b], acc_sh.at[dsts_v[b]], ssem, add=True))
    for b in range(nbuf):
      scatters[b].wait()

  @pl.loop(0, NGROUPS)
  def _(j):
    do_group(base0 + j * (NBUF * CHUNK), NBUF)

  for t in range(NTAIL):
    do_group(base0 + (NGROUPS * NBUF + t) * CHUNK, 1)

  if REM:
    base = base0 + NCHUNKS * CHUNK
    pltpu.sync_copy(src_hbm.at[pl.ds(base, REM)], srcT)
    pltpu.sync_copy(dst_hbm.at[pl.ds(base, REM)], dstT)
    pltpu.async_copy(table_hbm.at[srcT], rowsT, gsem).wait()
    pltpu.sync_copy(rowsT, acc_sh.at[dstT], add=True)

  plsc.subcore_barrier()
  pltpu.sync_copy(acc_sh.at[pl.ds(row0, ROWS_PER_SUB)],
                  out_hbm.at[c, pl.ds(row0, ROWS_PER_SUB)])


@functools.partial(
    pl.kernel,
    out_type=jax.ShapeDtypeStruct((NW, N_PAD), jnp.float32),
    mesh=_sc_mesh,
    compiler_params=_sc_cp,
    scratch_types=[
        pltpu.VMEM((EPW,), jnp.int32),
        pltpu.VMEM((N_PAD,), jnp.float32),
    ],
)
def _sc_degree(dst_hbm, out_hbm, dst_v, acc_v):
  c = lax.axis_index("c")
  s = lax.axis_index("s")
  wid = c * NS + s

  zeros = jnp.zeros((NL,), jnp.float32)

  @pl.loop(0, N_PAD // NL)
  def _(j):
    acc_v[pl.ds(j * NL, NL)] = zeros

  pltpu.sync_copy(dst_hbm.at[pl.ds(wid * EPW, EPW)], dst_v)

  ones = jnp.ones((NL,), jnp.float32)

  @pl.loop(0, EPW // NL)
  def _(i):
    idx = dst_v[pl.ds(i * NL, NL)]
    plsc.addupdate_scatter(acc_v, [idx], ones)

  pltpu.sync_copy(acc_v, out_hbm.at[wid])


@functools.partial(
    pl.kernel,
    out_type=jax.ShapeDtypeStruct((NW, N_PAD), jnp.float32),
    mesh=_sc_mesh,
    compiler_params=_sc_cp,
    scratch_types=[
        pltpu.VMEM((EPW,), jnp.int32),
        pltpu.VMEM((EPW,), jnp.int32),
        pltpu.VMEM((N_PAD,), jnp.float32),
        pltpu.VMEM((N_PAD,), jnp.float32),
    ],
)
def _sc_agg_scalar(src_hbm, dst_hbm, table_hbm, out_hbm,
                   src_v, dst_v, tab_v, acc_v):
  c = lax.axis_index("c")
  s = lax.axis_index("s")
  wid = c * NS + s

  zeros = jnp.zeros((NL,), jnp.float32)

  @pl.loop(0, N_PAD // NL)
  def _(j):
    acc_v[pl.ds(j * NL, NL)] = zeros

  pltpu.sync_copy(table_hbm, tab_v)
  pltpu.sync_copy(src_hbm.at[pl.ds(wid * EPW, EPW)], src_v)
  pltpu.sync_copy(dst_hbm.at[pl.ds(wid * EPW, EPW)], dst_v)

  @pl.loop(0, EPW // NL)
  def _(i):
    si = src_v[pl.ds(i * NL, NL)]
    di = dst_v[pl.ds(i * NL, NL)]
    val = plsc.load_gather(tab_v, [si])
    plsc.addupdate_scatter(acc_v, [di], val)

  pltpu.sync_copy(acc_v, out_hbm.at[wid])


_RB = 1024
_GRID = N_PAD // _RB


def _tc_matmul_body(x_ref, w_ref, o_ref):
  o_ref[...] = jnp.dot(x_ref[...], w_ref[...],
                       preferred_element_type=jnp.float32)


def _tc_matmul(x, w):
  return pl.pallas_call(
      _tc_matmul_body,
      grid=(_GRID,),
      in_specs=[
          pl.BlockSpec((_RB, DIM), lambda i: (i, 0)),
          pl.BlockSpec((DIM, DIM), lambda i: (0, 0)),
      ],
      out_specs=pl.BlockSpec((_RB, DIM), lambda i: (i, 0)),
      out_shape=jax.ShapeDtypeStruct((N_PAD, DIM), jnp.float32),
  )(x, w)


def _tc_scale_body(deg_ref, h_ref, h1p_ref, dinv_ref):
  deg = jnp.sum(deg_ref[...], axis=0) + 1.0
  dinv = lax.rsqrt(deg)
  h1p_ref[...] = h_ref[...] * dinv[:, None]
  dinv_ref[...] = dinv[:, None]


def _tc_scale(deg_parts, h):
  return pl.pallas_call(
      _tc_scale_body,
      grid=(_GRID,),
      in_specs=[
          pl.BlockSpec((NW, _RB), lambda i: (0, i)),
          pl.BlockSpec((_RB, DIM), lambda i: (i, 0)),
      ],
      out_specs=[
          pl.BlockSpec((_RB, DIM), lambda i: (i, 0)),
          pl.BlockSpec((_RB, 1), lambda i: (i, 0)),
      ],
      out_shape=[
          jax.ShapeDtypeStruct((N_PAD, DIM), jnp.float32),
          jax.ShapeDtypeStruct((N_PAD, 1), jnp.float32),
      ],
  )(deg_parts, h)


def _tc_layer1_body(p_ref, h1p_ref, dinv_ref, b1_ref, w2_ref, h2s_ref):
  dinv = dinv_ref[...]
  agg = p_ref[0] + p_ref[1] + h1p_ref[...]
  z1 = jnp.maximum(dinv * agg + b1_ref[...], 0.0)
  h2 = jnp.dot(z1, w2_ref[...], preferred_element_type=jnp.float32)
  h2s_ref[...] = dinv * h2


def _tc_layer1(parts, h1p, dinv, b1, w2):
  return pl.pallas_call(
      _tc_layer1_body,
      grid=(_GRID,),
      in_specs=[
          pl.BlockSpec((NC, _RB, DIM), lambda i: (0, i, 0)),
          pl.BlockSpec((_RB, DIM), lambda i: (i, 0)),
          pl.BlockSpec((_RB, 1), lambda i: (i, 0)),
          pl.BlockSpec((1, DIM), lambda i: (0, 0)),
          pl.BlockSpec((DIM, 1), lambda i: (0, 0)),
      ],
      out_specs=pl.BlockSpec((_RB, 1), lambda i: (i, 0)),
      out_shape=jax.ShapeDtypeStruct((N_PAD, 1), jnp.float32),
  )(parts, h1p, dinv, b1, w2)


def _tc_final_body(q_ref, h2s_ref, dinv_ref, b2_ref, o_ref):
  qsum = jnp.sum(q_ref[...], axis=0)[:, None]
  o_ref[...] = dinv_ref[...] * (qsum + h2s_ref[...]) + b2_ref[...]


def _tc_final(qparts, h2s, dinv, b2):
  return pl.pallas_call(
      _tc_final_body,
      grid=(_GRID,),
      in_specs=[
          pl.BlockSpec((NW, _RB), lambda i: (0, i)),
          pl.BlockSpec((_RB, 1), lambda i: (i, 0)),
          pl.BlockSpec((_RB, 1), lambda i: (i, 0)),
          pl.BlockSpec((1, 1), lambda i: (0, 0)),
      ],
      out_specs=pl.BlockSpec((_RB, 1), lambda i: (i, 0)),
      out_shape=jax.ShapeDtypeStruct((N_PAD, 1), jnp.float32),
  )(qparts, h2s, dinv, b2)


@jax.jit
def kernel(x, edge_index, W1, b1, W2, b2):
  src = edge_index[0].astype(jnp.int32)
  dst = edge_index[1].astype(jnp.int32)
  x = jnp.pad(x, ((0, N_PAD - N_NODES), (0, 0)))

  zeros128 = jnp.zeros((N_PAD, DIM), jnp.float32)

  deg_parts = _sc_degree(dst)
  h = _tc_matmul(x, W1)

  h1p, dinv = _tc_scale(deg_parts, h)

  parts = _sc_agg128(src, dst, h1p, zeros128)

  h2s = _tc_layer1(parts, h1p, dinv, b1.reshape(1, DIM), W2)

  qparts = _sc_agg_scalar(src, dst, h2s.reshape(N_PAD))

  out = _tc_final(qparts, h2s, dinv, b2.reshape(1, 1))
  return out[:N_NODES]


# speedup vs baseline: 44.5917x; 1.0099x over previous
import dataclasses
import functools

import jax
import jax.numpy as jnp
from jax import lax
from jax.experimental import pallas as pl
from jax.experimental.pallas import tpu as pltpu
from jax.experimental.pallas import tpu_sc as plsc

N_NODES = 10000
N_PAD = 10240
N_EDGES = 320000
DIM = 128

NC = 2
NS = 16
NL = 16
NW = NC * NS
EPW = N_EDGES // NW
CHUNK = 80
NCHUNKS = EPW // CHUNK
REM = EPW % CHUNK
ROWS_PER_SUB = N_PAD // NS

_sc_mesh = plsc.VectorSubcoreMesh(core_axis_name="c", subcore_axis_name="s")

_sc_cp = pltpu.CompilerParams()
if "needs_layout_passes" in pltpu.CompilerParams.__dataclass_fields__:
  _sc_cp = dataclasses.replace(_sc_cp, needs_layout_passes=False)


NBUF = 4
NGROUPS = NCHUNKS // NBUF
NTAIL = NCHUNKS % NBUF


@functools.partial(
    pl.kernel,
    out_type=jax.ShapeDtypeStruct((NC, N_PAD, DIM), jnp.float32),
    mesh=_sc_mesh,
    scratch_types=(
        [pltpu.VMEM((CHUNK,), jnp.int32) for _ in range(2 * NBUF)]
        + [pltpu.VMEM((CHUNK, DIM), jnp.float32) for _ in range(NBUF)]
        + ([pltpu.VMEM((REM,), jnp.int32),
            pltpu.VMEM((REM,), jnp.int32),
            pltpu.VMEM((REM, DIM), jnp.float32)] if REM else [])
        + [
            pltpu.VMEM_SHARED((N_PAD, DIM), jnp.float32),
            pltpu.SemaphoreType.DMA,
            pltpu.SemaphoreType.DMA,
            pltpu.SemaphoreType.DMA,
        ]
    ),
)
def _sc_agg128(src_hbm, dst_hbm, table_hbm, zeros_hbm, out_hbm, *refs):
  srcs_v = refs[0:NBUF]
  dsts_v = refs[NBUF:2 * NBUF]
  rows_v = refs[2 * NBUF:3 * NBUF]
  if REM:
    srcT, dstT, rowsT = refs[3 * NBUF:3 * NBUF + 3]
  acc_sh, isem, gsem, ssem = refs[-4:]
  c = lax.axis_index("c")
  s = lax.axis_index("s")
  wid = c * NS + s
  row0 = s * ROWS_PER_SUB
  pltpu.sync_copy(zeros_hbm.at[pl.ds(row0, ROWS_PER_SUB)],
                  acc_sh.at[pl.ds(row0, ROWS_PER_SUB)])
  plsc.subcore_barrier()

  base0 = wid * EPW

  def do_group(base, nbuf):
    icopies = []
    for b in range(nbuf):
      icopies.append(
          (pltpu.async_copy(src_hbm.at[pl.ds(base + b * CHUNK, CHUNK)],
                            srcs_v[b], isem),
           pltpu.async_copy(dst_hbm.at[pl.ds(base + b * CHUNK, CHUNK)],
                            dsts_v[b], isem)))
    gathers = []
    for b in range(nbuf):
      icopies[b][0].wait()
      icopies[b][1].wait()
      gathers.append(pltpu.async_copy(
          table_hbm.at[srcs_v[b]], rows_v[b], gsem))
    scatters = []
    for b in range(nbuf):
      gathers[b].wait()
      scatters.append(pltpu.async_copy(
          rows_v[b], acc_sh.at[dsts_v[b]], ssem, add=True))
    for b in range(nbuf):
      scatters[b].wait()

  @pl.loop(0, NGROUPS)
  def _(j):
    do_group(base0 + j * (NBUF * CHUNK), NBUF)

  for t in range(NTAIL):
    do_group(base0 + (NGROUPS * NBUF + t) * CHUNK, 1)

  if REM:
    base = base0 + NCHUNKS * CHUNK
    pltpu.sync_copy(src_hbm.at[pl.ds(base, REM)], srcT)
    pltpu.sync_copy(dst_hbm.at[pl.ds(base, REM)], dstT)
    pltpu.async_copy(table_hbm.at[srcT], rowsT, gsem).wait()
    pltpu.sync_copy(rowsT, acc_sh.at[dstT], add=True)

  plsc.subcore_barrier()
  pltpu.sync_copy(acc_sh.at[pl.ds(row0, ROWS_PER_SUB)],
                  out_hbm.at[c, pl.ds(row0, ROWS_PER_SUB)])


@functools.partial(
    pl.kernel,
    out_type=jax.ShapeDtypeStruct((NW, N_PAD), jnp.float32),
    mesh=_sc_mesh,
    compiler_params=_sc_cp,
    scratch_types=[
        pltpu.VMEM((EPW,), jnp.int32),
        pltpu.VMEM((N_PAD,), jnp.float32),
    ],
)
def _sc_degree(dst_hbm, out_hbm, dst_v, acc_v):
  c = lax.axis_index("c")
  s = lax.axis_index("s")
  wid = c * NS + s

  zeros = jnp.zeros((NL,), jnp.float32)

  @pl.loop(0, N_PAD // NL)
  def _(j):
    acc_v[pl.ds(j * NL, NL)] = zeros

  pltpu.sync_copy(dst_hbm.at[pl.ds(wid * EPW, EPW)], dst_v)

  ones = jnp.ones((NL,), jnp.float32)

  @pl.loop(0, EPW // NL)
  def _(i):
    idx = dst_v[pl.ds(i * NL, NL)]
    plsc.addupdate_scatter(acc_v, [idx], ones)

  pltpu.sync_copy(acc_v, out_hbm.at[wid])


@functools.partial(
    pl.kernel,
    out_type=jax.ShapeDtypeStruct((NW, N_PAD), jnp.float32),
    mesh=_sc_mesh,
    compiler_params=_sc_cp,
    scratch_types=[
        pltpu.VMEM((EPW,), jnp.int32),
        pltpu.VMEM((EPW,), jnp.int32),
        pltpu.VMEM((N_PAD,), jnp.float32),
        pltpu.VMEM((N_PAD,), jnp.float32),
    ],
)
def _sc_agg_scalar(src_hbm, dst_hbm, table_hbm, out_hbm,
                   src_v, dst_v, tab_v, acc_v):
  c = lax.axis_index("c")
  s = lax.axis_index("s")
  wid = c * NS + s

  zeros = jnp.zeros((NL,), jnp.float32)

  @pl.loop(0, N_PAD // NL)
  def _(j):
    acc_v[pl.ds(j * NL, NL)] = zeros

  pltpu.sync_copy(table_hbm, tab_v)
  pltpu.sync_copy(src_hbm.at[pl.ds(wid * EPW, EPW)], src_v)
  pltpu.sync_copy(dst_hbm.at[pl.ds(wid * EPW, EPW)], dst_v)

  @pl.loop(0, EPW // NL)
  def _(i):
    si = src_v[pl.ds(i * NL, NL)]
    di = dst_v[pl.ds(i * NL, NL)]
    val = plsc.load_gather(tab_v, [si])
    plsc.addupdate_scatter(acc_v, [di], val)

  pltpu.sync_copy(acc_v, out_hbm.at[wid])


_RB = 1024
_GRID = N_PAD // _RB


def _tc_matmul_body(x_ref, w_ref, o_ref):
  o_ref[...] = jnp.dot(x_ref[...], w_ref[...],
                       preferred_element_type=jnp.float32)


def _tc_matmul(x, w):
  return pl.pallas_call(
      _tc_matmul_body,
      grid=(_GRID,),
      in_specs=[
          pl.BlockSpec((_RB, DIM), lambda i: (i, 0)),
          pl.BlockSpec((DIM, DIM), lambda i: (0, 0)),
      ],
      out_specs=pl.BlockSpec((_RB, DIM), lambda i: (i, 0)),
      out_shape=jax.ShapeDtypeStruct((N_PAD, DIM), jnp.float32),
  )(x, w)


def _tc_scale_body(deg_ref, x_ref, w_ref, h1p_ref, dinv_ref):
  deg = jnp.sum(deg_ref[...], axis=0) + 1.0
  dinv = lax.rsqrt(deg)
  h = jnp.dot(x_ref[...], w_ref[...], preferred_element_type=jnp.float32)
  h1p_ref[...] = h * dinv[:, None]
  dinv_ref[...] = dinv[:, None]


def _tc_scale(deg_parts, x, w):
  return pl.pallas_call(
      _tc_scale_body,
      grid=(_GRID,),
      in_specs=[
          pl.BlockSpec((NW, _RB), lambda i: (0, i)),
          pl.BlockSpec((_RB, DIM), lambda i: (i, 0)),
          pl.BlockSpec((DIM, DIM), lambda i: (0, 0)),
      ],
      out_specs=[
          pl.BlockSpec((_RB, DIM), lambda i: (i, 0)),
          pl.BlockSpec((_RB, 1), lambda i: (i, 0)),
      ],
      out_shape=[
          jax.ShapeDtypeStruct((N_PAD, DIM), jnp.float32),
          jax.ShapeDtypeStruct((N_PAD, 1), jnp.float32),
      ],
  )(deg_parts, x, w)


def _tc_layer1_body(p_ref, h1p_ref, dinv_ref, b1_ref, w2_ref, h2s_ref):
  dinv = dinv_ref[...]
  agg = p_ref[0] + p_ref[1] + h1p_ref[...]
  z1 = jnp.maximum(dinv * agg + b1_ref[...], 0.0)
  h2 = jnp.dot(z1, w2_ref[...], preferred_element_type=jnp.float32)
  h2s_ref[...] = dinv * h2


def _tc_layer1(parts, h1p, dinv, b1, w2):
  return pl.pallas_call(
      _tc_layer1_body,
      grid=(_GRID,),
      in_specs=[
          pl.BlockSpec((NC, _RB, DIM), lambda i: (0, i, 0)),
          pl.BlockSpec((_RB, DIM), lambda i: (i, 0)),
          pl.BlockSpec((_RB, 1), lambda i: (i, 0)),
          pl.BlockSpec((1, DIM), lambda i: (0, 0)),
          pl.BlockSpec((DIM, 1), lambda i: (0, 0)),
      ],
      out_specs=pl.BlockSpec((_RB, 1), lambda i: (i, 0)),
      out_shape=jax.ShapeDtypeStruct((N_PAD, 1), jnp.float32),
  )(parts, h1p, dinv, b1, w2)


def _tc_final_body(q_ref, h2s_ref, dinv_ref, b2_ref, o_ref):
  qsum = jnp.sum(q_ref[...], axis=0)[:, None]
  o_ref[...] = dinv_ref[...] * (qsum + h2s_ref[...]) + b2_ref[...]


def _tc_final(qparts, h2s, dinv, b2):
  return pl.pallas_call(
      _tc_final_body,
      grid=(_GRID,),
      in_specs=[
          pl.BlockSpec((NW, _RB), lambda i: (0, i)),
          pl.BlockSpec((_RB, 1), lambda i: (i, 0)),
          pl.BlockSpec((_RB, 1), lambda i: (i, 0)),
          pl.BlockSpec((1, 1), lambda i: (0, 0)),
      ],
      out_specs=pl.BlockSpec((_RB, 1), lambda i: (i, 0)),
      out_shape=jax.ShapeDtypeStruct((N_PAD, 1), jnp.float32),
  )(qparts, h2s, dinv, b2)


@jax.jit
def kernel(x, edge_index, W1, b1, W2, b2):
  src = edge_index[0].astype(jnp.int32)
  dst = edge_index[1].astype(jnp.int32)
  x = jnp.pad(x, ((0, N_PAD - N_NODES), (0, 0)))

  zeros128 = jnp.zeros((N_PAD, DIM), jnp.float32)

  deg_parts = _sc_degree(dst)
  h1p, dinv = _tc_scale(deg_parts, x, W1)

  parts = _sc_agg128(src, dst, h1p, zeros128)

  h2s = _tc_layer1(parts, h1p, dinv, b1.reshape(1, DIM), W2)

  qparts = _sc_agg_scalar(src, dst, h2s.reshape(N_PAD))

  out = _tc_final(qparts, h2s, dinv, b2.reshape(1, 1))
  return out[:N_NODES]


# speedup vs baseline: 44.6043x; 1.0003x over previous
import dataclasses
import functools

import jax
import jax.numpy as jnp
from jax import lax
from jax.experimental import pallas as pl
from jax.experimental.pallas import tpu as pltpu
from jax.experimental.pallas import tpu_sc as plsc

N_NODES = 10000
N_PAD = 10240
N_EDGES = 320000
DIM = 128

NC = 2
NS = 16
NL = 16
NW = NC * NS
EPW = N_EDGES // NW
CHUNK = 80
NCHUNKS = EPW // CHUNK
REM = EPW % CHUNK
ROWS_PER_SUB = N_PAD // NS

_sc_mesh = plsc.VectorSubcoreMesh(core_axis_name="c", subcore_axis_name="s")

_sc_cp = pltpu.CompilerParams()
if "needs_layout_passes" in pltpu.CompilerParams.__dataclass_fields__:
  _sc_cp = dataclasses.replace(_sc_cp, needs_layout_passes=False)


NBUF = 4
NGROUPS = NCHUNKS // NBUF
NTAIL = NCHUNKS % NBUF


@functools.partial(
    pl.kernel,
    out_type=jax.ShapeDtypeStruct((NC, N_PAD, DIM), jnp.float32),
    mesh=_sc_mesh,
    scratch_types=(
        [pltpu.VMEM((CHUNK,), jnp.int32) for _ in range(2 * NBUF)]
        + [pltpu.VMEM((CHUNK, DIM), jnp.float32) for _ in range(NBUF)]
        + ([pltpu.VMEM((REM,), jnp.int32),
            pltpu.VMEM((REM,), jnp.int32),
            pltpu.VMEM((REM, DIM), jnp.float32)] if REM else [])
        + [
            pltpu.VMEM_SHARED((N_PAD, DIM), jnp.float32),
            pltpu.SemaphoreType.DMA,
            pltpu.SemaphoreType.DMA,
            pltpu.SemaphoreType.DMA,
        ]
    ),
)
def _sc_agg128(src_hbm, dst_hbm, table_hbm, zeros_hbm, out_hbm, *refs):
  srcs_v = refs[0:NBUF]
  dsts_v = refs[NBUF:2 * NBUF]
  rows_v = refs[2 * NBUF:3 * NBUF]
  if REM:
    srcT, dstT, rowsT = refs[3 * NBUF:3 * NBUF + 3]
  acc_sh, isem, gsem, ssem = refs[-4:]
  c = lax.axis_index("c")
  s = lax.axis_index("s")
  wid = c * NS + s
  row0 = s * ROWS_PER_SUB
  pltpu.sync_copy(zeros_hbm.at[pl.ds(row0, ROWS_PER_SUB)],
                  acc_sh.at[pl.ds(row0, ROWS_PER_SUB)])
  plsc.subcore_barrier()

  base0 = wid * EPW

  def do_group(base, nbuf):
    icopies = []
    for b in range(nbuf):
      icopies.append(
          (pltpu.async_copy(src_hbm.at[pl.ds(base + b * CHUNK, CHUNK)],
                            srcs_v[b], isem),
           pltpu.async_copy(dst_hbm.at[pl.ds(base + b * CHUNK, CHUNK)],
                            dsts_v[b], isem)))
    gathers = []
    for b in range(nbuf):
      icopies[b][0].wait()
      icopies[b][1].wait()
      gathers.append(pltpu.async_copy(
          table_hbm.at[srcs_v[b]], rows_v[b], gsem))
    scatters = []
    for b in range(nbuf):
      gathers[b].wait()
      scatters.append(pltpu.async_copy(
          rows_v[b], acc_sh.at[dsts_v[b]], ssem, add=True))
    for b in range(nbuf):
      scatters[b].wait()

  @pl.loop(0, NGROUPS)
  def _(j):
    do_group(base0 + j * (NBUF * CHUNK), NBUF)

  for t in range(NTAIL):
    do_group(base0 + (NGROUPS * NBUF + t) * CHUNK, 1)

  if REM:
    base = base0 + NCHUNKS * CHUNK
    pltpu.sync_copy(src_hbm.at[pl.ds(base, REM)], srcT)
    pltpu.sync_copy(dst_hbm.at[pl.ds(base, REM)], dstT)
    pltpu.async_copy(table_hbm.at[srcT], rowsT, gsem).wait()
    pltpu.sync_copy(rowsT, acc_sh.at[dstT], add=True)

  plsc.subcore_barrier()
  pltpu.sync_copy(acc_sh.at[pl.ds(row0, ROWS_PER_SUB)],
                  out_hbm.at[c, pl.ds(row0, ROWS_PER_SUB)])


@functools.partial(
    pl.kernel,
    out_type=jax.ShapeDtypeStruct((NW, N_PAD), jnp.float32),
    mesh=_sc_mesh,
    compiler_params=_sc_cp,
    scratch_types=[
        pltpu.VMEM((EPW,), jnp.int32),
        pltpu.VMEM((N_PAD,), jnp.float32),
    ],
)
def _sc_degree(dst_hbm, out_hbm, dst_v, acc_v):
  c = lax.axis_index("c")
  s = lax.axis_index("s")
  wid = c * NS + s

  zeros = jnp.zeros((NL,), jnp.float32)

  @pl.loop(0, N_PAD // NL)
  def _(j):
    acc_v[pl.ds(j * NL, NL)] = zeros

  pltpu.sync_copy(dst_hbm.at[pl.ds(wid * EPW, EPW)], dst_v)

  ones = jnp.ones((NL,), jnp.float32)

  @pl.loop(0, EPW // NL)
  def _(i):
    idx = dst_v[pl.ds(i * NL, NL)]
    plsc.addupdate_scatter(acc_v, [idx], ones)

  pltpu.sync_copy(acc_v, out_hbm.at[wid])


@functools.partial(
    pl.kernel,
    out_type=jax.ShapeDtypeStruct((NW, N_PAD), jnp.float32),
    mesh=_sc_mesh,
    compiler_params=_sc_cp,
    scratch_types=[
        pltpu.VMEM((EPW,), jnp.int32),
        pltpu.VMEM((EPW,), jnp.int32),
        pltpu.VMEM((N_PAD,), jnp.float32),
        pltpu.VMEM((N_PAD,), jnp.float32),
    ],
)
def _sc_agg_scalar(src_hbm, dst_hbm, table_hbm, out_hbm,
                   src_v, dst_v, tab_v, acc_v):
  c = lax.axis_index("c")
  s = lax.axis_index("s")
  wid = c * NS + s

  zeros = jnp.zeros((NL,), jnp.float32)

  @pl.loop(0, N_PAD // NL)
  def _(j):
    acc_v[pl.ds(j * NL, NL)] = zeros

  pltpu.sync_copy(table_hbm, tab_v)
  pltpu.sync_copy(src_hbm.at[pl.ds(wid * EPW, EPW)], src_v)
  pltpu.sync_copy(dst_hbm.at[pl.ds(wid * EPW, EPW)], dst_v)

  @pl.loop(0, EPW // NL)
  def _(i):
    si = src_v[pl.ds(i * NL, NL)]
    di = dst_v[pl.ds(i * NL, NL)]
    val = plsc.load_gather(tab_v, [si])
    plsc.addupdate_scatter(acc_v, [di], val)

  pltpu.sync_copy(acc_v, out_hbm.at[wid])


_RB = 1024
_GRID = N_PAD // _RB


def _tc_scale_body(deg_ref, x_ref, w_ref, h1p_ref, dinv_ref):
  deg = jnp.sum(deg_ref[...], axis=0) + 1.0
  dinv = lax.rsqrt(deg)
  h = jnp.dot(x_ref[...], w_ref[...], preferred_element_type=jnp.float32)
  h1p_ref[...] = h * dinv[:, None]
  dinv_ref[...] = dinv[:, None]


def _tc_scale(deg_parts, x, w):
  return pl.pallas_call(
      _tc_scale_body,
      grid=(_GRID,),
      in_specs=[
          pl.BlockSpec((NW, _RB), lambda i: (0, i)),
          pl.BlockSpec((_RB, DIM), lambda i: (i, 0)),
          pl.BlockSpec((DIM, DIM), lambda i: (0, 0)),
      ],
      out_specs=[
          pl.BlockSpec((_RB, DIM), lambda i: (i, 0)),
          pl.BlockSpec((_RB, 1), lambda i: (i, 0)),
      ],
      out_shape=[
          jax.ShapeDtypeStruct((N_PAD, DIM), jnp.float32),
          jax.ShapeDtypeStruct((N_PAD, 1), jnp.float32),
      ],
  )(deg_parts, x, w)


def _tc_layer1_body(p_ref, h1p_ref, dinv_ref, b1_ref, w2_ref, h2s_ref):
  dinv = dinv_ref[...]
  agg = p_ref[0] + p_ref[1] + h1p_ref[...]
  z1 = jnp.maximum(dinv * agg + b1_ref[...], 0.0)
  h2 = jnp.dot(z1, w2_ref[...], preferred_element_type=jnp.float32)
  h2s_ref[...] = dinv * h2


def _tc_layer1(parts, h1p, dinv, b1, w2):
  return pl.pallas_call(
      _tc_layer1_body,
      grid=(_GRID,),
      in_specs=[
          pl.BlockSpec((NC, _RB, DIM), lambda i: (0, i, 0)),
          pl.BlockSpec((_RB, DIM), lambda i: (i, 0)),
          pl.BlockSpec((_RB, 1), lambda i: (i, 0)),
          pl.BlockSpec((1, DIM), lambda i: (0, 0)),
          pl.BlockSpec((DIM, 1), lambda i: (0, 0)),
      ],
      out_specs=pl.BlockSpec((_RB, 1), lambda i: (i, 0)),
      out_shape=jax.ShapeDtypeStruct((N_PAD, 1), jnp.float32),
  )(parts, h1p, dinv, b1, w2)


def _tc_final_body(q_ref, h2s_ref, dinv_ref, b2_ref, o_ref):
  qsum = jnp.sum(q_ref[...], axis=0)[:, None]
  o_ref[...] = dinv_ref[...] * (qsum + h2s_ref[...]) + b2_ref[...]


def _tc_final(qparts, h2s, dinv, b2):
  return pl.pallas_call(
      _tc_final_body,
      grid=(_GRID,),
      in_specs=[
          pl.BlockSpec((NW, _RB), lambda i: (0, i)),
          pl.BlockSpec((_RB, 1), lambda i: (i, 0)),
          pl.BlockSpec((_RB, 1), lambda i: (i, 0)),
          pl.BlockSpec((1, 1), lambda i: (0, 0)),
      ],
      out_specs=pl.BlockSpec((_RB, 1), lambda i: (i, 0)),
      out_shape=jax.ShapeDtypeStruct((N_PAD, 1), jnp.float32),
  )(qparts, h2s, dinv, b2)


@jax.jit
def kernel(x, edge_index, W1, b1, W2, b2):
  src = edge_index[0].astype(jnp.int32)
  dst = edge_index[1].astype(jnp.int32)
  x = jnp.pad(x, ((0, N_PAD - N_NODES), (0, 0)))

  zeros128 = jnp.zeros((N_PAD, DIM), jnp.float32)

  deg_parts = _sc_degree(dst)
  h1p, dinv = _tc_scale(deg_parts, x, W1)

  parts = _sc_agg128(src, dst, h1p, zeros128)

  h2s = _tc_layer1(parts, h1p, dinv, b1.reshape(1, DIM), W2)

  qparts = _sc_agg_scalar(src, dst, h2s.reshape(N_PAD))

  out = _tc_final(qparts, h2s, dinv, b2.reshape(1, 1))
  return out[:N_NODES]
